# baseline (speedup 1.0000x reference)
import functools

import jax
import jax.numpy as jnp
from jax import lax
from jax.experimental import pallas as pl
from jax.experimental.pallas import tpu as pltpu
from jax.experimental.pallas import tpu_sc as plsc

N = 10000
E = 320000
DIN = 128
DH = 128
DOUT = 40

NPAD = 10240
D1P = 48
ROW0 = DH + 16
ROW1 = D1P + 16

NC = 2
NS = 16
NW = NC * NS
EALL = E + N
EPAD = 331776
RB = 256
GDEPTH = 3
SDEPTH = 2


def _ones_tail(rb):
    return (lax.broadcasted_iota(jnp.int32, (rb, 16), 1) == 0).astype(jnp.float32)


def _tc1_body(x_ref, w0l_ref, b0l_ref, w0r_ref, b0r_ref, xn_ref, xl_ref, xr_ref):
    x = x_ref[...]
    nrm = jnp.sqrt(jnp.sum(x * x, axis=1, keepdims=True))
    xn = x / jnp.maximum(nrm, 1e-12)
    xn_ref[...] = xn
    mm = jnp.dot(xn, w0l_ref[...], preferred_element_type=jnp.float32) + b0l_ref[...]
    xl_ref[...] = jnp.concatenate([mm, _ones_tail(mm.shape[0])], axis=1)
    xr_ref[...] = jnp.dot(xn, w0r_ref[...], preferred_element_type=jnp.float32) + b0r_ref[...]


def _tc2_body(s0_ref, s1_ref, xn_ref, g0_ref, bt0_ref, bias0_ref,
              w1l_ref, b1l_ref, w1r_ref, b1r_ref, hl_ref, hr_ref):
    s = s0_ref[...] + s1_ref[...]
    num = s[:, :DH]
    den = s[:, DH:DH + 1]
    x1 = num / (den + 1e-16) + bias0_ref[...]
    mu = jnp.mean(x1, axis=1, keepdims=True)
    var = jnp.mean((x1 - mu) ** 2, axis=1, keepdims=True)
    x1 = (x1 - mu) / jnp.sqrt(var + 1e-5) * g0_ref[...] + bt0_ref[...]
    h = jnp.maximum(x1, 0.0) + xn_ref[...]
    mm = jnp.dot(h, w1l_ref[...], preferred_element_type=jnp.float32) + b1l_ref[...]
    hl_ref[...] = jnp.concatenate([mm, _ones_tail(mm.shape[0])], axis=1)
    hr_ref[...] = jnp.dot(h, w1r_ref[...], preferred_element_type=jnp.float32) + b1r_ref[...]


def _tc3_body(s0_ref, s1_ref, bias1_ref, out_ref):
    s = s0_ref[...] + s1_ref[...]
    den = s[:, D1P:D1P + 1]
    out_ref[...] = s / (den + 1e-16) + bias1_ref[...]


def _row_spec(w):
    return pl.BlockSpec((RB, w), lambda i: (i, 0))


def _full_spec(shape):
    return pl.BlockSpec(shape, lambda i: tuple(0 for _ in shape))


_GRID = NPAD // RB


def _tc1(xp, w0l, b0l, w0r, b0r):
    return pl.pallas_call(
        _tc1_body,
        grid=(_GRID,),
        in_specs=[_row_spec(DIN), _full_spec((DIN, DH)), _full_spec((1, DH)),
                  _full_spec((DIN, DH)), _full_spec((1, DH))],
        out_specs=[_row_spec(DIN), _row_spec(ROW0), _row_spec(DH)],
        out_shape=[jax.ShapeDtypeStruct((NPAD, DIN), jnp.float32),
                   jax.ShapeDtypeStruct((NPAD, ROW0), jnp.float32),
                   jax.ShapeDtypeStruct((NPAD, DH), jnp.float32)],
    )(xp, w0l, b0l.reshape(1, DH), w0r, b0r.reshape(1, DH))


def _tc2(s0, s1, xn, g0, bt0, bias0, w1l, b1l, w1r, b1r):
    return pl.pallas_call(
        _tc2_body,
        grid=(_GRID,),
        in_specs=[_row_spec(ROW0), _row_spec(ROW0), _row_spec(DIN),
                  _full_spec((1, DH)), _full_spec((1, DH)), _full_spec((1, DH)),
                  _full_spec((DH, D1P)), _full_spec((1, D1P)),
                  _full_spec((DH, D1P)), _full_spec((1, D1P))],
        out_specs=[_row_spec(ROW1), _row_spec(D1P)],
        out_shape=[jax.ShapeDtypeStruct((NPAD, ROW1), jnp.float32),
                   jax.ShapeDtypeStruct((NPAD, D1P), jnp.float32)],
    )(s0, s1, xn, g0.reshape(1, DH), bt0.reshape(1, DH), bias0.reshape(1, DH),
      w1l, b1l.reshape(1, D1P), w1r, b1r.reshape(1, D1P))


def _tc3(s0, s1, bias1):
    return pl.pallas_call(
        _tc3_body,
        grid=(_GRID,),
        in_specs=[_row_spec(ROW1), _row_spec(ROW1), _full_spec((1, ROW1))],
        out_specs=_row_spec(ROW1),
        out_shape=jax.ShapeDtypeStruct((NPAD, ROW1), jnp.float32),
    )(s0, s1, bias1.reshape(1, ROW1))


def _make_sc_edge(d, roww, k):
    mesh = plsc.VectorSubcoreMesh(core_axis_name="c", subcore_axis_name="s",
                                  num_cores=NC, num_subcores=NS)
    rpt = NPAD // NS
    chunks = EPAD // (NW * k)
    assert chunks % (GDEPTH * SDEPTH) == 0 and chunks >= 2 * GDEPTH

    scratch = [
        pltpu.VMEM_SHARED((NPAD, roww), jnp.float32),
        [pltpu.VMEM((k,), jnp.int32) for _ in range(GDEPTH)],
        [pltpu.VMEM((k,), jnp.int32) for _ in range(GDEPTH)],
        [pltpu.VMEM((k, roww), jnp.float32) for _ in range(GDEPTH)],
        [pltpu.VMEM((k, d), jnp.float32) for _ in range(GDEPTH)],
        [pltpu.VMEM((k, roww), jnp.float32) for _ in range(SDEPTH)],
        [pltpu.VMEM((k,), jnp.int32) for _ in range(SDEPTH)],
        pltpu.VMEM((d,), jnp.float32),
        [pltpu.SemaphoreType.DMA for _ in range(GDEPTH)],
        [pltpu.SemaphoreType.DMA for _ in range(GDEPTH)],
        [pltpu.SemaphoreType.DMA for _ in range(GDEPTH)],
        [pltpu.SemaphoreType.DMA for _ in range(SDEPTH)],
    ]

    @functools.partial(
        pl.kernel,
        out_type=jax.ShapeDtypeStruct((NC, NPAD, roww), jnp.float32),
        mesh=mesh,
        compiler_params=pltpu.CompilerParams(needs_layout_passes=False,
                                             use_tc_tiling_on_sc=False),
        scratch_types=scratch,
    )
    def body(xl, xr, src, dst, att, out, acc, idx_s, idx_d,
             rows_l, rows_r, rows_w, idx_sc, att_v, sem_ix, sem_l, sem_r, sem_sc):
        cid = lax.axis_index("c")
        sid = lax.axis_index("s")
        wid = sid * NC + cid
        ebase = wid * chunks * k
        pltpu.sync_copy(att, att_v)

        def issue_idx(ci, sl):
            base = pl.multiple_of(ebase + ci * k, k)
            pltpu.async_copy(src.at[pl.ds(base, k)], idx_s[sl], sem_ix[sl])
            pltpu.async_copy(dst.at[pl.ds(base, k)], idx_d[sl], sem_ix[sl])

        def wait_idx(sl):
            pltpu.make_async_copy(src.at[pl.ds(0, k)], idx_s[sl], sem_ix[sl]).wait()
            pltpu.make_async_copy(dst.at[pl.ds(0, k)], idx_d[sl], sem_ix[sl]).wait()

        def issue_gather(sl):
            pltpu.async_copy(xl.at[idx_s[sl]], rows_l[sl], sem_l[sl])
            pltpu.async_copy(xr.at[idx_d[sl]], rows_r[sl], sem_r[sl])

        def wait_gather(sl):
            pltpu.make_async_copy(xl.at[idx_s[sl]], rows_l[sl], sem_l[sl]).wait()
            pltpu.make_async_copy(xr.at[idx_d[sl]], rows_r[sl], sem_r[sl]).wait()

        def wait_scatter(st):
            pltpu.make_async_copy(rows_w[st], acc.at[idx_sc[st]], sem_sc[st]).wait()

        for c0 in range(GDEPTH - 1):
            issue_idx(c0, c0)
        wait_idx(0)
        issue_gather(0)

        def zrow(i, carry):
            for j in range(roww // 16):
                rows_w[0][i, pl.ds(j * 16, 16)] = jnp.zeros((16,), jnp.float32)
            return carry
        lax.fori_loop(0, k, zrow, 0)
        for b in range(rpt // k):
            pltpu.async_copy(rows_w[0], acc.at[pl.ds(sid * rpt + b * k, k)],
                             sem_sc[0])
        for b in range(rpt // k):
            pltpu.make_async_copy(rows_w[0],
                                  acc.at[pl.ds(sid * rpt, k)],
                                  sem_sc[0]).wait()
        plsc.subcore_barrier()

        def round_body(g, carry):
            i0 = g * (GDEPTH * SDEPTH)
            for u in range(GDEPTH * SDEPTH):
                i = i0 + u
                s = u % GDEPTH
                t = u % SDEPTH
                sp = (u + GDEPTH - 1) % GDEPTH
                sq = (u + 1) % GDEPTH
                @pl.when(i + GDEPTH - 1 < chunks)
                def _():
                    issue_idx(i + GDEPTH - 1, sp)
                @pl.when(i + 1 < chunks)
                def _():
                    wait_idx(sq)
                    issue_gather(sq)
                wait_gather(s)
                @pl.when(i >= SDEPTH)
                def _():
                    wait_scatter(t)

                @plsc.parallel_loop(0, k, unroll=4)
                def edge(e):
                    ls = [rows_l[s][e, pl.ds(j * 16, 16)]
                          for j in range(roww // 16)]
                    accv = jnp.zeros((16,), jnp.float32)
                    for j in range(d // 16):
                        z = ls[j] + rows_r[s][e, pl.ds(j * 16, 16)]
                        accv = accv + (jnp.where(z > 0, z, 0.2 * z)
                                       * att_v[pl.ds(j * 16, 16)])
                    wv = jnp.exp(jnp.broadcast_to(jnp.sum(accv), (16,)))
                    for j in range(roww // 16):
                        rows_w[t][e, pl.ds(j * 16, 16)] = ls[j] * wv

                for j in range(k // 16):
                    idx_sc[t][pl.ds(j * 16, 16)] = idx_d[s][pl.ds(j * 16, 16)]
                pltpu.async_copy(rows_w[t], acc.at[idx_sc[t]], sem_sc[t],
                                 add=True)
            return carry
        lax.fori_loop(0, chunks // (GDEPTH * SDEPTH), round_body, 0)
        for t in range(SDEPTH):
            wait_scatter(t)

        plsc.subcore_barrier()
        pltpu.sync_copy(acc.at[pl.ds(sid * rpt, rpt)],
                        out.at[cid, pl.ds(sid * rpt, rpt)])

    return body


_make_sc_edge = functools.cache(_make_sc_edge)


def kernel(x, edge_index, W0l, b0l, W0r, b0r, att0, bias0, g0, bt0,
           W1l, b1l, W1r, b1r, att1, bias1):
    xp = jnp.pad(x, ((0, NPAD - N), (0, 0)))
    loop = jnp.arange(N, dtype=jnp.int32)
    padi = jnp.full((EPAD - EALL,), N, dtype=jnp.int32)
    src = jnp.concatenate([edge_index[0].astype(jnp.int32), loop, padi])
    dst = jnp.concatenate([edge_index[1].astype(jnp.int32), loop, padi])

    xn, xl0, xr0 = _tc1(xp, W0l, b0l, W0r, b0r)
    p = _make_sc_edge(DH, ROW0, 32)(xl0, xr0, src, dst, att0)
    hl, hr = _tc2(p[0], p[1], xn, g0, bt0, bias0,
                  jnp.pad(W1l, ((0, 0), (0, D1P - DOUT))),
                  jnp.pad(b1l, (0, D1P - DOUT)),
                  jnp.pad(W1r, ((0, 0), (0, D1P - DOUT))),
                  jnp.pad(b1r, (0, D1P - DOUT)))
    q = _make_sc_edge(D1P, ROW1, 64)(hl, hr, src, dst, jnp.pad(att1, (0, D1P - DOUT)))
    o = _tc3(q[0], q[1], jnp.pad(bias1, (0, ROW1 - DOUT)))
    return o[:N, :DOUT]


# speedup vs baseline: 1.0293x; 1.0293x over previous
import functools

import jax
import jax.numpy as jnp
from jax import lax
from jax.experimental import pallas as pl
from jax.experimental.pallas import tpu as pltpu
from jax.experimental.pallas import tpu_sc as plsc

N = 10000
E = 320000
DIN = 128
DH = 128
DOUT = 40

NPAD = 10240
D1P = 48
ROW0 = DH + 16
ROW1 = D1P + 16

NC = 2
NS = 16
NW = NC * NS
EALL = E + N
EPAD = 331776
RB = 256
GDEPTH = 3
SDEPTH = 2


def _ones_tail(rb):
    return (lax.broadcasted_iota(jnp.int32, (rb, 16), 1) == 0).astype(jnp.float32)


def _tc1_body(x_ref, w0l_ref, b0l_ref, w0r_ref, b0r_ref, xn_ref, xl_ref, xr_ref):
    x = x_ref[...]
    nrm = jnp.sqrt(jnp.sum(x * x, axis=1, keepdims=True))
    xn = x / jnp.maximum(nrm, 1e-12)
    xn_ref[...] = xn
    mm = jnp.dot(xn, w0l_ref[...], preferred_element_type=jnp.float32) + b0l_ref[...]
    xl_ref[...] = jnp.concatenate([mm, _ones_tail(mm.shape[0])], axis=1)
    xr_ref[...] = jnp.dot(xn, w0r_ref[...], preferred_element_type=jnp.float32) + b0r_ref[...]


def _tc2_body(s0_ref, s1_ref, xn_ref, g0_ref, bt0_ref, bias0_ref,
              w1l_ref, b1l_ref, w1r_ref, b1r_ref, hl_ref, hr_ref):
    s = s0_ref[...] + s1_ref[...]
    num = s[:, :DH]
    den = s[:, DH:DH + 1]
    x1 = num / (den + 1e-16) + bias0_ref[...]
    mu = jnp.mean(x1, axis=1, keepdims=True)
    var = jnp.mean((x1 - mu) ** 2, axis=1, keepdims=True)
    x1 = (x1 - mu) / jnp.sqrt(var + 1e-5) * g0_ref[...] + bt0_ref[...]
    h = jnp.maximum(x1, 0.0) + xn_ref[...]
    mm = jnp.dot(h, w1l_ref[...], preferred_element_type=jnp.float32) + b1l_ref[...]
    hl_ref[...] = jnp.concatenate([mm, _ones_tail(mm.shape[0])], axis=1)
    hr_ref[...] = jnp.dot(h, w1r_ref[...], preferred_element_type=jnp.float32) + b1r_ref[...]


def _tc3_body(s0_ref, s1_ref, bias1_ref, out_ref):
    s = s0_ref[...] + s1_ref[...]
    den = s[:, D1P:D1P + 1]
    out_ref[...] = s / (den + 1e-16) + bias1_ref[...]


def _row_spec(w):
    return pl.BlockSpec((RB, w), lambda i: (i, 0))


def _full_spec(shape):
    return pl.BlockSpec(shape, lambda i: tuple(0 for _ in shape))


_GRID = NPAD // RB


def _tc1(xp, w0l, b0l, w0r, b0r):
    return pl.pallas_call(
        _tc1_body,
        grid=(_GRID,),
        in_specs=[_row_spec(DIN), _full_spec((DIN, DH)), _full_spec((1, DH)),
                  _full_spec((DIN, DH)), _full_spec((1, DH))],
        out_specs=[_row_spec(DIN), _row_spec(ROW0), _row_spec(DH)],
        out_shape=[jax.ShapeDtypeStruct((NPAD, DIN), jnp.float32),
                   jax.ShapeDtypeStruct((NPAD, ROW0), jnp.float32),
                   jax.ShapeDtypeStruct((NPAD, DH), jnp.float32)],
    )(xp, w0l, b0l.reshape(1, DH), w0r, b0r.reshape(1, DH))


def _tc2(s0, s1, xn, g0, bt0, bias0, w1l, b1l, w1r, b1r):
    return pl.pallas_call(
        _tc2_body,
        grid=(_GRID,),
        in_specs=[_row_spec(ROW0), _row_spec(ROW0), _row_spec(DIN),
                  _full_spec((1, DH)), _full_spec((1, DH)), _full_spec((1, DH)),
                  _full_spec((DH, D1P)), _full_spec((1, D1P)),
                  _full_spec((DH, D1P)), _full_spec((1, D1P))],
        out_specs=[_row_spec(ROW1), _row_spec(D1P)],
        out_shape=[jax.ShapeDtypeStruct((NPAD, ROW1), jnp.float32),
                   jax.ShapeDtypeStruct((NPAD, D1P), jnp.float32)],
    )(s0, s1, xn, g0.reshape(1, DH), bt0.reshape(1, DH), bias0.reshape(1, DH),
      w1l, b1l.reshape(1, D1P), w1r, b1r.reshape(1, D1P))


def _tc3(s0, s1, bias1):
    return pl.pallas_call(
        _tc3_body,
        grid=(_GRID,),
        in_specs=[_row_spec(ROW1), _row_spec(ROW1), _full_spec((1, ROW1))],
        out_specs=_row_spec(ROW1),
        out_shape=jax.ShapeDtypeStruct((NPAD, ROW1), jnp.float32),
    )(s0, s1, bias1.reshape(1, ROW1))


def _make_sc_edge(d, roww, k):
    mesh = plsc.VectorSubcoreMesh(core_axis_name="c", subcore_axis_name="s",
                                  num_cores=NC, num_subcores=NS)
    rpt = NPAD // NS
    chunks = EPAD // (NW * k)
    assert chunks % (GDEPTH * SDEPTH) == 0 and chunks >= 2 * GDEPTH

    scratch = [
        pltpu.VMEM_SHARED((NPAD, roww), jnp.float32),
        [pltpu.VMEM((k,), jnp.int32) for _ in range(GDEPTH)],
        [pltpu.VMEM((k,), jnp.int32) for _ in range(GDEPTH)],
        [pltpu.VMEM((k, roww), jnp.float32) for _ in range(GDEPTH)],
        [pltpu.VMEM((k, d), jnp.float32) for _ in range(GDEPTH)],
        [pltpu.VMEM((k, roww), jnp.float32) for _ in range(SDEPTH)],
        [pltpu.VMEM((k,), jnp.int32) for _ in range(SDEPTH)],
        pltpu.VMEM((d,), jnp.float32),
        [pltpu.SemaphoreType.DMA for _ in range(GDEPTH)],
        [pltpu.SemaphoreType.DMA for _ in range(GDEPTH)],
        [pltpu.SemaphoreType.DMA for _ in range(GDEPTH)],
        [pltpu.SemaphoreType.DMA for _ in range(SDEPTH)],
    ]

    @functools.partial(
        pl.kernel,
        out_type=jax.ShapeDtypeStruct((NC, NPAD, roww), jnp.float32),
        mesh=mesh,
        compiler_params=pltpu.CompilerParams(needs_layout_passes=False,
                                             use_tc_tiling_on_sc=False),
        scratch_types=scratch,
    )
    def body(xl, xr, src, dst, att, out, acc, idx_s, idx_d,
             rows_l, rows_r, rows_w, idx_sc, att_v, sem_ix, sem_l, sem_r, sem_sc):
        cid = lax.axis_index("c")
        sid = lax.axis_index("s")
        wid = sid * NC + cid
        ebase = wid * chunks * k
        pltpu.sync_copy(att, att_v)

        def issue_idx(ci, sl):
            base = pl.multiple_of(ebase + ci * k, k)
            pltpu.async_copy(src.at[pl.ds(base, k)], idx_s[sl], sem_ix[sl])
            pltpu.async_copy(dst.at[pl.ds(base, k)], idx_d[sl], sem_ix[sl])

        def wait_idx(sl):
            pltpu.make_async_copy(src.at[pl.ds(0, k)], idx_s[sl], sem_ix[sl]).wait()
            pltpu.make_async_copy(dst.at[pl.ds(0, k)], idx_d[sl], sem_ix[sl]).wait()

        def issue_gather(sl):
            pltpu.async_copy(xl.at[idx_s[sl]], rows_l[sl], sem_l[sl])
            pltpu.async_copy(xr.at[idx_d[sl]], rows_r[sl], sem_r[sl])

        def wait_gather(sl):
            pltpu.make_async_copy(xl.at[idx_s[sl]], rows_l[sl], sem_l[sl]).wait()
            pltpu.make_async_copy(xr.at[idx_d[sl]], rows_r[sl], sem_r[sl]).wait()

        def wait_scatter(st):
            pltpu.make_async_copy(rows_w[st], acc.at[idx_sc[st]], sem_sc[st]).wait()

        for c0 in range(GDEPTH - 1):
            issue_idx(c0, c0)
        wait_idx(0)
        issue_gather(0)

        def zrow(i, carry):
            for j in range(roww // 16):
                rows_w[0][i, pl.ds(j * 16, 16)] = jnp.zeros((16,), jnp.float32)
            return carry
        lax.fori_loop(0, k, zrow, 0)
        for b in range(rpt // k):
            pltpu.async_copy(rows_w[0], acc.at[pl.ds(sid * rpt + b * k, k)],
                             sem_sc[0])
        for b in range(rpt // k):
            pltpu.make_async_copy(rows_w[0],
                                  acc.at[pl.ds(sid * rpt, k)],
                                  sem_sc[0]).wait()
        plsc.subcore_barrier()

        def round_body(g, carry):
            i0 = g * (GDEPTH * SDEPTH)
            for u in range(GDEPTH * SDEPTH):
                i = i0 + u
                s = u % GDEPTH
                t = u % SDEPTH
                sp = (u + GDEPTH - 1) % GDEPTH
                sq = (u + 1) % GDEPTH
                @pl.when(i + GDEPTH - 1 < chunks)
                def _():
                    issue_idx(i + GDEPTH - 1, sp)
                @pl.when(i + 1 < chunks)
                def _():
                    wait_idx(sq)
                    issue_gather(sq)
                wait_gather(s)
                @pl.when(i >= SDEPTH)
                def _():
                    wait_scatter(t)

                @plsc.parallel_loop(0, k, unroll=8)
                def edge(e):
                    ls = [rows_l[s][e, pl.ds(j * 16, 16)]
                          for j in range(roww // 16)]
                    accv = jnp.zeros((16,), jnp.float32)
                    for j in range(d // 16):
                        z = ls[j] + rows_r[s][e, pl.ds(j * 16, 16)]
                        accv = accv + (jnp.where(z > 0, z, 0.2 * z)
                                       * att_v[pl.ds(j * 16, 16)])
                    wv = jnp.exp(jnp.broadcast_to(jnp.sum(accv), (16,)))
                    for j in range(roww // 16):
                        rows_w[t][e, pl.ds(j * 16, 16)] = ls[j] * wv

                for j in range(k // 16):
                    idx_sc[t][pl.ds(j * 16, 16)] = idx_d[s][pl.ds(j * 16, 16)]
                pltpu.async_copy(rows_w[t], acc.at[idx_sc[t]], sem_sc[t],
                                 add=True)
            return carry
        lax.fori_loop(0, chunks // (GDEPTH * SDEPTH), round_body, 0)
        for t in range(SDEPTH):
            wait_scatter(t)

        plsc.subcore_barrier()
        pltpu.sync_copy(acc.at[pl.ds(sid * rpt, rpt)],
                        out.at[cid, pl.ds(sid * rpt, rpt)])

    return body


_make_sc_edge = functools.cache(_make_sc_edge)


def kernel(x, edge_index, W0l, b0l, W0r, b0r, att0, bias0, g0, bt0,
           W1l, b1l, W1r, b1r, att1, bias1):
    xp = jnp.pad(x, ((0, NPAD - N), (0, 0)))
    loop = jnp.arange(N, dtype=jnp.int32)
    padi = jnp.full((EPAD - EALL,), N, dtype=jnp.int32)
    src = jnp.concatenate([edge_index[0].astype(jnp.int32), loop, padi])
    dst = jnp.concatenate([edge_index[1].astype(jnp.int32), loop, padi])

    xn, xl0, xr0 = _tc1(xp, W0l, b0l, W0r, b0r)
    p = _make_sc_edge(DH, ROW0, 32)(xl0, xr0, src, dst, att0)
    hl, hr = _tc2(p[0], p[1], xn, g0, bt0, bias0,
                  jnp.pad(W1l, ((0, 0), (0, D1P - DOUT))),
                  jnp.pad(b1l, (0, D1P - DOUT)),
                  jnp.pad(W1r, ((0, 0), (0, D1P - DOUT))),
                  jnp.pad(b1r, (0, D1P - DOUT)))
    q = _make_sc_edge(D1P, ROW1, 64)(hl, hr, src, dst, jnp.pad(att1, (0, D1P - DOUT)))
    o = _tc3(q[0], q[1], jnp.pad(bias1, (0, ROW1 - DOUT)))
    return o[:N, :DOUT]
